# baseline (speedup 1.0000x reference)
import functools

import jax
import jax.numpy as jnp
from jax import lax
from jax.experimental import pallas as pl
from jax.experimental.pallas import tpu as pltpu
from jax.experimental.pallas import tpu_sc as plsc

VOCAB = 100000
EMBED = 64
BATCH = 4096
SEQ = 200
B_TOTAL = BATCH * SEQ

_info = plsc.get_sparse_core_info()
NC, NS = _info.num_cores, _info.num_subcores
NW = NC * NS
B_PER_W = B_TOTAL // NW
CHUNK = 128
N_CHUNKS = B_PER_W // CHUNK
NBUF = 4
INFLIGHT = 2
N_OUTER = N_CHUNKS // NBUF
UNROLL = 4

_mesh = plsc.VectorSubcoreMesh(core_axis_name="c", subcore_axis_name="s")


@functools.partial(
    pl.kernel,
    mesh=_mesh,
    out_type=jax.ShapeDtypeStruct((B_TOTAL, EMBED), jnp.float32),
    scratch_types=[
        pltpu.VMEM((N_CHUNKS, CHUNK), jnp.int32),
        pltpu.VMEM((NBUF, CHUNK, EMBED), jnp.float32),
        pltpu.SemaphoreType.DMA,
        pltpu.SemaphoreType.DMA,
    ],
    compiler_params=pltpu.CompilerParams(use_tc_tiling_on_sc=False),
)
def _gather_kernel(table_hbm, idx_hbm, out_hbm, idx_v, rows_v, gsem, wsem):
    wid = lax.axis_index("s") * NC + lax.axis_index("c")
    base = wid * B_PER_W

    pltpu.sync_copy(idx_hbm.at[wid], idx_v)

    def gather_start(chunk, slot):
        def body(i, _):
            v = idx_v[chunk, pl.ds(i * 16, 16)]
            for u in range(16):
                pltpu.async_copy(
                    table_hbm.at[pl.ds(v[u], 1)],
                    rows_v.at[slot].at[pl.ds(i * 16 + u, 1)],
                    gsem,
                )
            return 0

        lax.fori_loop(0, CHUNK // 16, body, 0)

    def gather_wait():
        pltpu.make_async_copy(
            table_hbm.at[pl.ds(0, CHUNK)], rows_v.at[0], gsem
        ).wait()

    def wb_start(chunk, slot):
        pltpu.async_copy(
            rows_v.at[slot],
            out_hbm.at[pl.ds(base + chunk * CHUNK, CHUNK)],
            wsem,
        )

    def wb_wait():
        pltpu.make_async_copy(
            rows_v.at[0], out_hbm.at[pl.ds(base, CHUNK)], wsem
        ).wait()

    for b in range(INFLIGHT):
        gather_start(b, b)

    def outer(o, carry):
        for b in range(NBUF):
            g = o * NBUF + b
            gather_wait()
            wb_start(g, b)
            if b < INFLIGHT:
                @pl.when(o > 0)
                def _():
                    wb_wait()

                gather_start(g + INFLIGHT, b + INFLIGHT)
            else:
                wb_wait()

                @pl.when(o < N_OUTER - 1)
                def _():
                    gather_start(g + INFLIGHT, (b + INFLIGHT) % NBUF)
        return carry

    lax.fori_loop(0, N_OUTER, outer, 0)

    for _ in range(INFLIGHT):
        wb_wait()


def kernel(x, table):
    idx = x.reshape(NW, N_CHUNKS, CHUNK)
    out = _gather_kernel(table, idx)
    return out.reshape(BATCH, SEQ, EMBED)


# speedup vs baseline: 1.0141x; 1.0141x over previous
import functools

import jax
import jax.numpy as jnp
from jax import lax
from jax.experimental import pallas as pl
from jax.experimental.pallas import tpu as pltpu
from jax.experimental.pallas import tpu_sc as plsc

VOCAB = 100000
EMBED = 64
BATCH = 4096
SEQ = 200
B_TOTAL = BATCH * SEQ

_info = plsc.get_sparse_core_info()
NC, NS = _info.num_cores, _info.num_subcores
NW = NC * NS
B_PER_W = B_TOTAL // NW
CHUNK = 128
STREAM_ROWS = 64
DMA_ROWS = CHUNK - STREAM_ROWS
N_CHUNKS = B_PER_W // CHUNK
NBUF = 8
INFLIGHT = 4
N_OUTER = N_CHUNKS // NBUF

_mesh = plsc.VectorSubcoreMesh(core_axis_name="c", subcore_axis_name="s")


@functools.partial(
    pl.kernel,
    mesh=_mesh,
    out_type=jax.ShapeDtypeStruct((B_TOTAL, EMBED), jnp.float32),
    scratch_types=[
        pltpu.VMEM((N_CHUNKS, CHUNK), jnp.int32),
        pltpu.VMEM((NBUF, CHUNK, EMBED), jnp.float32),
        pltpu.SemaphoreType.DMA,
        pltpu.SemaphoreType.DMA,
        pltpu.SemaphoreType.DMA,
    ],
    compiler_params=pltpu.CompilerParams(use_tc_tiling_on_sc=False),
)
def _gather_kernel(table_hbm, idx_hbm, out_hbm, idx_v, rows_v, gsem, dsem, wsem):
    wid = lax.axis_index("s") * NC + lax.axis_index("c")
    base = wid * B_PER_W

    pltpu.sync_copy(idx_hbm.at[wid], idx_v)

    def gather_start(chunk, slot):
        pltpu.async_copy(
            table_hbm.at[idx_v.at[chunk].at[pl.ds(0, STREAM_ROWS)]],
            rows_v.at[slot].at[pl.ds(0, STREAM_ROWS)],
            gsem,
        )

        def body(i, _):
            v = idx_v[chunk, pl.ds(STREAM_ROWS + i * 16, 16)]
            for u in range(16):
                pltpu.async_copy(
                    table_hbm.at[pl.ds(v[u], 1)],
                    rows_v.at[slot].at[pl.ds(STREAM_ROWS + i * 16 + u, 1)],
                    dsem,
                )
            return 0

        lax.fori_loop(0, DMA_ROWS // 16, body, 0)

    def gather_wait():
        pltpu.make_async_copy(
            table_hbm.at[idx_v.at[0].at[pl.ds(0, STREAM_ROWS)]],
            rows_v.at[0].at[pl.ds(0, STREAM_ROWS)],
            gsem,
        ).wait()
        pltpu.make_async_copy(
            table_hbm.at[pl.ds(0, DMA_ROWS)],
            rows_v.at[0].at[pl.ds(0, DMA_ROWS)],
            dsem,
        ).wait()

    def wb_start(chunk, slot):
        pltpu.async_copy(
            rows_v.at[slot],
            out_hbm.at[pl.ds(base + chunk * CHUNK, CHUNK)],
            wsem,
        )

    def wb_wait():
        pltpu.make_async_copy(
            rows_v.at[0], out_hbm.at[pl.ds(base, CHUNK)], wsem
        ).wait()

    for b in range(INFLIGHT):
        gather_start(b, b)

    def outer(o, carry):
        for b in range(NBUF):
            g = o * NBUF + b
            gather_wait()
            wb_start(g, b)
            if b < INFLIGHT:
                @pl.when(o > 0)
                def _():
                    wb_wait()

                gather_start(g + INFLIGHT, b + INFLIGHT)
            else:
                wb_wait()

                @pl.when(o < N_OUTER - 1)
                def _():
                    gather_start(g + INFLIGHT, (b + INFLIGHT) % NBUF)
        return carry

    lax.fori_loop(0, N_OUTER, outer, 0)

    for _ in range(INFLIGHT):
        wb_wait()


def kernel(x, table):
    idx = x.reshape(NW, N_CHUNKS, CHUNK)
    out = _gather_kernel(table, idx)
    return out.reshape(BATCH, SEQ, EMBED)
